# baseline (speedup 1.0000x reference)
import functools

import jax
import jax.numpy as jnp
from jax import lax
from jax.experimental import pallas as pl
from jax.experimental.pallas import tpu as pltpu
from jax.experimental.pallas import tpu_sc as plsc

Q = 1024
QC = 512
Q_TILE = 128
N_TILE = 2048
N_PAD = 20480
N_REAL = 20000
BIG = 1.0e30
T0 = 3.0e38

NC, NS, L = 2, 16, 16
NW = NC * NS
ROWS_PER_W = QC // NW
VREGS_PER_ROW = N_PAD // L
CAND = 224
COMPACT_AT = 80
BLK = 8
SUP = 64


def _dist_kernel(state_ref, buf_ref, w1_ref, b1_ref, w2_ref, b2_ref,
                 w3_ref, b3_ref, r_ref, neural_ref):
    j = pl.program_id(1)

    s = state_ref[...]
    b = buf_ref[...]

    prod = lax.dot_general(s, b, (((1,), (1,)), ((), ())),
                           preferred_element_type=jnp.float32)
    ones = jnp.ones((1, b.shape[1]), jnp.float32)
    bn = lax.dot_general(ones, b * b, (((1,), (1,)), ((), ())),
                         preferred_element_type=jnp.float32)
    qn = jnp.sum(s * s, axis=1, keepdims=True)
    d2 = (bn - 2.0 * prod) + qn
    r = jnp.sqrt(jnp.maximum(d2, 1e-12))

    col = lax.broadcasted_iota(jnp.int32, (Q_TILE, N_TILE), 1) + j * N_TILE
    r_ref[...] = jnp.where(col < N_REAL, r, T0)

    @pl.when(j == 0)
    def _mlp():
        h1 = jnp.maximum(
            lax.dot_general(s, w1_ref[...], (((1,), (0,)), ((), ())),
                            preferred_element_type=jnp.float32)
            + b1_ref[...], 0.0)
        h2 = jnp.maximum(
            lax.dot_general(h1, w2_ref[...], (((1,), (0,)), ((), ())),
                            preferred_element_type=jnp.float32)
            + b2_ref[...], 0.0)
        z = (lax.dot_general(h2, w3_ref[...], (((1,), (0,)), ((), ())),
                             preferred_element_type=jnp.float32)
             + b3_ref[...])
        neural_ref[...] = (1.0 - jax.nn.sigmoid(z[:, 0:1]))[:, 0]


def _distances_and_mlp(state, buf, w1, b1p, w2, b2p, w3, b3p):
    grid = (QC // Q_TILE, N_PAD // N_TILE)
    return pl.pallas_call(
        _dist_kernel,
        grid=grid,
        in_specs=[
            pl.BlockSpec((Q_TILE, 512), lambda i, j: (i, 0)),
            pl.BlockSpec((N_TILE, 512), lambda i, j: (j, 0)),
            pl.BlockSpec((512, 128), lambda i, j: (0, 0)),
            pl.BlockSpec((1, 128), lambda i, j: (0, 0)),
            pl.BlockSpec((128, 128), lambda i, j: (0, 0)),
            pl.BlockSpec((1, 128), lambda i, j: (0, 0)),
            pl.BlockSpec((128, 128), lambda i, j: (0, 0)),
            pl.BlockSpec((1, 128), lambda i, j: (0, 0)),
        ],
        out_specs=[
            pl.BlockSpec((Q_TILE, N_TILE), lambda i, j: (i, j)),
            pl.BlockSpec((Q_TILE,), lambda i, j: (i,)),
        ],
        out_shape=[
            jax.ShapeDtypeStruct((QC, N_PAD), jnp.float32),
            jax.ShapeDtypeStruct((QC,), jnp.float32),
        ],
        compiler_params=pltpu.CompilerParams(
            dimension_semantics=("arbitrary", "arbitrary")),
    )(state, buf, w1, b1p, w2, b2p, w3, b3p)


def _extract_sums(cand_vregs):
    def body(_, carry):
        prev, cnt, s10, s15, s20, t, cb = carry
        w = jnp.full((L,), T0, jnp.float32)
        for v in cand_vregs:
            w = jnp.minimum(w, jnp.where(v > prev, v, T0))
        m = jnp.min(w)
        c = jnp.float32(0.0)
        for v in cand_vregs:
            c = c + jnp.sum(jnp.where(v == m, 1.0, 0.0))
        new_cnt = cnt + c
        crossing = jnp.logical_and(cnt < 20.0, new_cnt >= 20.0)
        t = jnp.where(crossing, m, t)
        cb = jnp.where(crossing, cnt, cb)
        s10 = s10 + m * jnp.clip(10.0 - cnt, 0.0, c)
        s15 = s15 + m * jnp.clip(15.0 - cnt, 0.0, c)
        s20 = s20 + m * jnp.clip(20.0 - cnt, 0.0, c)
        return m, new_cnt, s10, s15, s20, t, cb

    z = jnp.float32(0.0)
    init = (jnp.float32(-T0), z, z, z, z, jnp.float32(T0), z)
    return lax.fori_loop(0, 20, body, init)


def _sc_topk_kernel(r_hbm, neural_hbm, out_hbm,
                    buf0, buf1, cand, neu_v, out_v, sem0, sem1):
    wid = lax.axis_index("s") * NC + lax.axis_index("c")
    base = wid * ROWS_PER_W
    iota = lax.iota(jnp.int32, L)

    pltpu.sync_copy(neural_hbm.at[pl.ds(base, ROWS_PER_W)], neu_v)

    def reset_cand():
        for vi in range(CAND // L):
            cand[pl.ds(vi * L, L)] = jnp.full((L,), T0, jnp.float32)

    def compact(t_p):
        del t_p
        cvs = [cand[pl.ds(vi * L, L)] for vi in range(CAND // L)]
        _, _, _, _, _, t, _ = _extract_sums(cvs)
        q = jnp.int32(0)
        for v in cvs:
            mk = v < t
            plsc.store_compressed(cand.at[pl.ds(q, L)], v, mask=mk)
            q = q + jnp.sum(mk.astype(jnp.int32))
        n_t = jnp.int32(20) - q
        tv = jnp.full((L,), t, jnp.float32)
        for blk in range(2):
            mk = (iota + blk * L) < n_t
            plsc.store_compressed(cand.at[pl.ds(q + blk * L, L)], tv, mask=mk)
        q = q + n_t
        for vi in range(CAND // L):
            slot = iota + vi * L
            v = cand[pl.ds(vi * L, L)]
            cand[pl.ds(vi * L, L)] = jnp.where(slot < q, v, T0)
        return t, q

    def sub_insert(sv, t_p):
        t, p = t_p
        mks = [v < t for v in sv]
        cs = [plsc.all_reduce_population_count(mk)[0] for mk in mks]
        offs = []
        o = p
        for c in cs:
            offs.append(o)
            o = o + c
        for v, mk, off in zip(sv, mks, offs):
            plsc.store_compressed(cand.at[pl.ds(off, L)], v, mask=mk)
        return lax.cond(o >= COMPACT_AT, compact, lambda tp: tp, (t, o))

    def scan_row(rowbuf):
        reset_cand()

        def tree_min(vs):
            ms = list(vs)
            while len(ms) > 1:
                ms = [jnp.minimum(ms[i], ms[i + 1])
                      for i in range(0, len(ms), 2)]
            return ms[0]

        def block_body(bi, t_p):
            base_v = bi * SUP
            accs = [rowbuf[pl.ds((base_v + u) * L, L)] for u in range(4)]
            for u in range(4, SUP):
                k = u & 3
                accs[k] = jnp.minimum(accs[k],
                                      rowbuf[pl.ds((base_v + u) * L, L)])
            mt = tree_min(accs)

            def slow(t_p):
                for u in range(SUP // BLK):
                    sv = [rowbuf[pl.ds((base_v + u * BLK + w) * L, L)]
                          for w in range(BLK)]
                    ms = tree_min(sv)
                    sub_hit = (plsc.all_reduce_population_count(
                        ms < t_p[0])[0] > 0)
                    t_p = lax.cond(sub_hit,
                                   functools.partial(sub_insert, sv),
                                   lambda tp: tp, t_p)
                return t_p

            hit = plsc.all_reduce_population_count(mt < t_p[0])[0] > 0
            return lax.cond(hit, slow, lambda tp: tp, t_p)

        lax.fori_loop(0, VREGS_PER_ROW // SUP, block_body,
                      (jnp.float32(T0), jnp.int32(0)))
        cvs = [cand[pl.ds(vi * L, L)] for vi in range(CAND // L)]
        _, _, s10, s15, s20, _, _ = _extract_sums(cvs)
        return (s10 * 0.1 + s15 * (1.0 / 15.0) + s20 * 0.05) * (1.0 / 3.0)

    def wait(buf, sem):
        pltpu.make_async_copy(r_hbm.at[0], buf, sem).wait()

    for g in range(ROWS_PER_W // L):
        g0 = g * L
        pltpu.async_copy(r_hbm.at[base + g0], buf0, sem0)
        pltpu.async_copy(r_hbm.at[base + g0 + 1], buf1, sem1)

        def pair_body(pi, acc, g0=g0):
            r0 = g0 + 2 * pi

            wait(buf0, sem0)
            ens0 = scan_row(buf0)

            @pl.when(pi < L // 2 - 1)
            def _pf0():
                pltpu.async_copy(r_hbm.at[base + r0 + 2], buf0, sem0)

            wait(buf1, sem1)
            ens1 = scan_row(buf1)

            @pl.when(pi < L // 2 - 1)
            def _pf1():
                pltpu.async_copy(r_hbm.at[base + r0 + 3], buf1, sem1)

            acc = jnp.where(iota == 2 * pi, ens0, acc)
            acc = jnp.where(iota == 2 * pi + 1, ens1, acc)
            return acc

        acc = lax.fori_loop(0, L // 2, pair_body,
                            jnp.zeros((L,), jnp.float32))
        neu = neu_v[pl.ds(g * L, L)]
        out_v[pl.ds(g * L, L)] = 0.7 * acc + 0.3 * neu

    pltpu.sync_copy(out_v, out_hbm.at[pl.ds(base, ROWS_PER_W)])


def _sc_topk(r, neural):
    mesh = plsc.VectorSubcoreMesh(core_axis_name="c", subcore_axis_name="s")
    f = pl.kernel(
        _sc_topk_kernel,
        out_type=jax.ShapeDtypeStruct((QC,), jnp.float32),
        mesh=mesh,
        scratch_types=[
            pltpu.VMEM((N_PAD,), jnp.float32),
            pltpu.VMEM((N_PAD,), jnp.float32),
            pltpu.VMEM((CAND,), jnp.float32),
            pltpu.VMEM((ROWS_PER_W,), jnp.float32),
            pltpu.VMEM((ROWS_PER_W,), jnp.float32),
            pltpu.SemaphoreType.DMA,
            pltpu.SemaphoreType.DMA,
        ],
        compiler_params=pltpu.CompilerParams(needs_layout_passes=False),
    )
    return f(r, neural)


@jax.jit
def kernel(state, state_buffer, W1, b1, W2, b2, W3, b3):
    N = state_buffer.shape[0]
    buf = jnp.pad(state_buffer, ((0, N_PAD - N), (0, 0)))
    b1p = b1[None, :]
    w2 = jnp.pad(W2, ((0, 0), (0, 128 - W2.shape[1])))
    b2p = jnp.pad(b2, (0, 128 - b2.shape[0]))[None, :]
    w3 = jnp.pad(W3, ((0, 128 - W3.shape[0]), (0, 128 - W3.shape[1])))
    b3p = jnp.pad(b3, (0, 128 - b3.shape[0]))[None, :]

    outs = []
    for c in range(Q // QC):
        sc = lax.slice_in_dim(state, c * QC, (c + 1) * QC, axis=0)
        r, neural = _distances_and_mlp(sc, buf, W1, b1p, w2, b2p, w3, b3p)
        outs.append(_sc_topk(r, neural))
    return jnp.concatenate(outs, axis=0)


# speedup vs baseline: 3.7737x; 3.7737x over previous
import functools

import jax
import jax.numpy as jnp
from jax import lax
from jax.experimental import pallas as pl
from jax.experimental.pallas import tpu as pltpu
from jax.experimental.pallas import tpu_sc as plsc

Q = 1024
QC = 512
Q_TILE = 128
N_TILE = 2048
N_PAD = 20480
N_REAL = 20000
BIG = 1.0e30
T0 = 3.0e38

NC, NS, L = 2, 16, 16
NW = NC * NS
ROWS_PER_W = QC // NW
VREGS_PER_ROW = N_PAD // L
CAND = 224
COMPACT_AT = 80
BLK = 8
GROUP = 16
NGROUPS = N_PAD // L // GROUP
NFOLD = NGROUPS // 16


def _dist_kernel(state_ref, buf_ref, w1_ref, b1_ref, w2_ref, b2_ref,
                 w3_ref, b3_ref, r_ref, neural_ref):
    j = pl.program_id(1)

    s = state_ref[...]
    b = buf_ref[...]

    prod = lax.dot_general(s, b, (((1,), (1,)), ((), ())),
                           preferred_element_type=jnp.float32)
    ones = jnp.ones((1, b.shape[1]), jnp.float32)
    bn = lax.dot_general(ones, b * b, (((1,), (1,)), ((), ())),
                         preferred_element_type=jnp.float32)
    qn = jnp.sum(s * s, axis=1, keepdims=True)
    d2 = (bn - 2.0 * prod) + qn
    r = jnp.sqrt(jnp.maximum(d2, 1e-12))

    col = lax.broadcasted_iota(jnp.int32, (Q_TILE, N_TILE), 1) + j * N_TILE
    r_ref[...] = jnp.where(col < N_REAL, r, T0)

    @pl.when(j == 0)
    def _mlp():
        h1 = jnp.maximum(
            lax.dot_general(s, w1_ref[...], (((1,), (0,)), ((), ())),
                            preferred_element_type=jnp.float32)
            + b1_ref[...], 0.0)
        h2 = jnp.maximum(
            lax.dot_general(h1, w2_ref[...], (((1,), (0,)), ((), ())),
                            preferred_element_type=jnp.float32)
            + b2_ref[...], 0.0)
        z = (lax.dot_general(h2, w3_ref[...], (((1,), (0,)), ((), ())),
                             preferred_element_type=jnp.float32)
             + b3_ref[...])
        neural_ref[...] = (1.0 - jax.nn.sigmoid(z[:, 0:1]))[:, 0]


def _distances_and_mlp(state, buf, w1, b1p, w2, b2p, w3, b3p):
    grid = (QC // Q_TILE, N_PAD // N_TILE)
    return pl.pallas_call(
        _dist_kernel,
        grid=grid,
        in_specs=[
            pl.BlockSpec((Q_TILE, 512), lambda i, j: (i, 0)),
            pl.BlockSpec((N_TILE, 512), lambda i, j: (j, 0)),
            pl.BlockSpec((512, 128), lambda i, j: (0, 0)),
            pl.BlockSpec((1, 128), lambda i, j: (0, 0)),
            pl.BlockSpec((128, 128), lambda i, j: (0, 0)),
            pl.BlockSpec((1, 128), lambda i, j: (0, 0)),
            pl.BlockSpec((128, 128), lambda i, j: (0, 0)),
            pl.BlockSpec((1, 128), lambda i, j: (0, 0)),
        ],
        out_specs=[
            pl.BlockSpec((Q_TILE, N_TILE), lambda i, j: (i, j)),
            pl.BlockSpec((Q_TILE,), lambda i, j: (i,)),
        ],
        out_shape=[
            jax.ShapeDtypeStruct((QC, N_PAD), jnp.float32),
            jax.ShapeDtypeStruct((QC,), jnp.float32),
        ],
        compiler_params=pltpu.CompilerParams(
            dimension_semantics=("arbitrary", "arbitrary")),
    )(state, buf, w1, b1p, w2, b2p, w3, b3p)


def _extract_sums(cand_vregs):
    def body(_, carry):
        prev, cnt, s10, s15, s20, t, cb = carry
        w = jnp.full((L,), T0, jnp.float32)
        for v in cand_vregs:
            w = jnp.minimum(w, jnp.where(v > prev, v, T0))
        m = jnp.min(w)
        c = jnp.float32(0.0)
        for v in cand_vregs:
            c = c + jnp.sum(jnp.where(v == m, 1.0, 0.0))
        new_cnt = cnt + c
        crossing = jnp.logical_and(cnt < 20.0, new_cnt >= 20.0)
        t = jnp.where(crossing, m, t)
        cb = jnp.where(crossing, cnt, cb)
        s10 = s10 + m * jnp.clip(10.0 - cnt, 0.0, c)
        s15 = s15 + m * jnp.clip(15.0 - cnt, 0.0, c)
        s20 = s20 + m * jnp.clip(20.0 - cnt, 0.0, c)
        return m, new_cnt, s10, s15, s20, t, cb

    z = jnp.float32(0.0)
    init = (jnp.float32(-T0), z, z, z, z, jnp.float32(T0), z)
    return lax.fori_loop(0, 20, body, init)


def _sc_topk_kernel(r_hbm, neural_hbm, out_hbm,
                    buf0, buf1, cand, bmbuf, neu_v, out_v, sem0, sem1):
    wid = lax.axis_index("s") * NC + lax.axis_index("c")
    base = wid * ROWS_PER_W
    iota = lax.iota(jnp.int32, L)

    pltpu.sync_copy(neural_hbm.at[pl.ds(base, ROWS_PER_W)], neu_v)

    def reset_cand():
        for vi in range(CAND // L):
            cand[pl.ds(vi * L, L)] = jnp.full((L,), T0, jnp.float32)

    def compact(t_p):
        del t_p
        cvs = [cand[pl.ds(vi * L, L)] for vi in range(CAND // L)]
        _, _, _, _, _, t, _ = _extract_sums(cvs)
        q = jnp.int32(0)
        for v in cvs:
            mk = v < t
            plsc.store_compressed(cand.at[pl.ds(q, L)], v, mask=mk)
            q = q + jnp.sum(mk.astype(jnp.int32))
        n_t = jnp.int32(20) - q
        tv = jnp.full((L,), t, jnp.float32)
        for blk in range(2):
            mk = (iota + blk * L) < n_t
            plsc.store_compressed(cand.at[pl.ds(q + blk * L, L)], tv, mask=mk)
        q = q + n_t
        for vi in range(CAND // L):
            slot = iota + vi * L
            v = cand[pl.ds(vi * L, L)]
            cand[pl.ds(vi * L, L)] = jnp.where(slot < q, v, T0)
        return t, q

    def sub_insert(sv, t_p):
        t, p = t_p
        mks = [v < t for v in sv]
        cs = [plsc.all_reduce_population_count(mk)[0] for mk in mks]
        offs = []
        o = p
        for c in cs:
            offs.append(o)
            o = o + c
        for v, mk, off in zip(sv, mks, offs):
            plsc.store_compressed(cand.at[pl.ds(off, L)], v, mask=mk)
        return lax.cond(o >= COMPACT_AT, compact, lambda tp: tp, (t, o))

    def tree_min(vs):
        ms = list(vs)
        while len(ms) > 1:
            ms = [jnp.minimum(ms[i], ms[i + 1])
                  for i in range(0, len(ms), 2)]
        return ms[0]

    def scan_row(rowbuf):
        reset_cand()

        def pass_a(qi, _):
            for gq in range(4):
                gi = qi * 4 + gq
                b0 = gi * GROUP
                a0 = rowbuf[pl.ds(b0 * L, L)]
                a1 = rowbuf[pl.ds((b0 + 1) * L, L)]
                for u in range(2, GROUP, 2):
                    a0 = jnp.minimum(a0, rowbuf[pl.ds((b0 + u) * L, L)])
                    a1 = jnp.minimum(a1, rowbuf[pl.ds((b0 + u + 1) * L, L)])
                bmbuf[pl.ds(gi * L, L)] = jnp.minimum(a0, a1)
            return 0
        lax.fori_loop(0, NGROUPS // 4, pass_a, 0)

        tms = []
        for h in range(NFOLD):
            acc = bmbuf[pl.ds(h * 16 * L, L)]
            for u in range(1, 16):
                acc = jnp.minimum(acc, bmbuf[pl.ds((h * 16 + u) * L, L)])
            tms.append(acc)
        _, _, _, _, _, t_ub, _ = _extract_sums(tms)
        t_plus = lax.bitcast_convert_type(
            lax.bitcast_convert_type(t_ub, jnp.int32) + 1, jnp.float32)

        def pass_b(gi, t_p):
            bm = bmbuf[pl.ds(gi * L, L)]
            hit = plsc.all_reduce_population_count(bm < t_p[0])[0] > 0

            def rescan(t_p):
                for u in range(GROUP // BLK):
                    sv = [rowbuf[pl.ds((gi * GROUP + u * BLK + w) * L, L)]
                          for w in range(BLK)]
                    ms = tree_min(sv)
                    sub_hit = (plsc.all_reduce_population_count(
                        ms < t_p[0])[0] > 0)
                    t_p = lax.cond(sub_hit,
                                   functools.partial(sub_insert, sv),
                                   lambda tp: tp, t_p)
                return t_p

            return lax.cond(hit, rescan, lambda tp: tp, t_p)

        lax.fori_loop(0, NGROUPS, pass_b, (t_plus, jnp.int32(0)))
        cvs = [cand[pl.ds(vi * L, L)] for vi in range(CAND // L)]
        _, _, s10, s15, s20, _, _ = _extract_sums(cvs)
        return (s10 * 0.1 + s15 * (1.0 / 15.0) + s20 * 0.05) * (1.0 / 3.0)

    def wait(buf, sem):
        pltpu.make_async_copy(r_hbm.at[0], buf, sem).wait()

    for g in range(ROWS_PER_W // L):
        g0 = g * L
        pltpu.async_copy(r_hbm.at[base + g0], buf0, sem0)
        pltpu.async_copy(r_hbm.at[base + g0 + 1], buf1, sem1)

        def pair_body(pi, acc, g0=g0):
            r0 = g0 + 2 * pi

            wait(buf0, sem0)
            ens0 = scan_row(buf0)

            @pl.when(pi < L // 2 - 1)
            def _pf0():
                pltpu.async_copy(r_hbm.at[base + r0 + 2], buf0, sem0)

            wait(buf1, sem1)
            ens1 = scan_row(buf1)

            @pl.when(pi < L // 2 - 1)
            def _pf1():
                pltpu.async_copy(r_hbm.at[base + r0 + 3], buf1, sem1)

            acc = jnp.where(iota == 2 * pi, ens0, acc)
            acc = jnp.where(iota == 2 * pi + 1, ens1, acc)
            return acc

        acc = lax.fori_loop(0, L // 2, pair_body,
                            jnp.zeros((L,), jnp.float32))
        neu = neu_v[pl.ds(g * L, L)]
        out_v[pl.ds(g * L, L)] = 0.7 * acc + 0.3 * neu

    pltpu.sync_copy(out_v, out_hbm.at[pl.ds(base, ROWS_PER_W)])


def _sc_topk(r, neural):
    mesh = plsc.VectorSubcoreMesh(core_axis_name="c", subcore_axis_name="s")
    f = pl.kernel(
        _sc_topk_kernel,
        out_type=jax.ShapeDtypeStruct((QC,), jnp.float32),
        mesh=mesh,
        scratch_types=[
            pltpu.VMEM((N_PAD,), jnp.float32),
            pltpu.VMEM((N_PAD,), jnp.float32),
            pltpu.VMEM((CAND,), jnp.float32),
            pltpu.VMEM((NGROUPS * L,), jnp.float32),
            pltpu.VMEM((ROWS_PER_W,), jnp.float32),
            pltpu.VMEM((ROWS_PER_W,), jnp.float32),
            pltpu.SemaphoreType.DMA,
            pltpu.SemaphoreType.DMA,
        ],
        compiler_params=pltpu.CompilerParams(needs_layout_passes=False),
    )
    return f(r, neural)


@jax.jit
def kernel(state, state_buffer, W1, b1, W2, b2, W3, b3):
    N = state_buffer.shape[0]
    buf = jnp.pad(state_buffer, ((0, N_PAD - N), (0, 0)))
    b1p = b1[None, :]
    w2 = jnp.pad(W2, ((0, 0), (0, 128 - W2.shape[1])))
    b2p = jnp.pad(b2, (0, 128 - b2.shape[0]))[None, :]
    w3 = jnp.pad(W3, ((0, 128 - W3.shape[0]), (0, 128 - W3.shape[1])))
    b3p = jnp.pad(b3, (0, 128 - b3.shape[0]))[None, :]

    outs = []
    for c in range(Q // QC):
        sc = lax.slice_in_dim(state, c * QC, (c + 1) * QC, axis=0)
        r, neural = _distances_and_mlp(sc, buf, W1, b1p, w2, b2p, w3, b3p)
        outs.append(_sc_topk(r, neural))
    return jnp.concatenate(outs, axis=0)


# speedup vs baseline: 3.8485x; 1.0198x over previous
import functools

import jax
import jax.numpy as jnp
from jax import lax
from jax.experimental import pallas as pl
from jax.experimental.pallas import tpu as pltpu
from jax.experimental.pallas import tpu_sc as plsc

Q = 1024
QC = 512
Q_TILE = 128
N_TILE = 2048
N_PAD = 20480
N_REAL = 20000
BIG = 1.0e30
T0 = 3.0e38

NC, NS, L = 2, 16, 16
NW = NC * NS
ROWS_PER_W = QC // NW
VREGS_PER_ROW = N_PAD // L
CAND = 224
COMPACT_AT = 80
BLK = 8
GROUP = 16
NGROUPS = N_PAD // L // GROUP
NFOLD = NGROUPS // 16


def _dist_kernel(state2_ref, buf_ref, w1_ref, b1_ref, w2_ref, b2_ref,
                 w3_ref, b3_ref, r_ref, neural_ref, bn_ref):
    i = pl.program_id(0)
    j = pl.program_id(1)

    s2 = state2_ref[...]
    b = buf_ref[...]

    prod2 = lax.dot_general(s2, b, (((1,), (1,)), ((), ())),
                            preferred_element_type=jnp.float32)

    @pl.when(i == 0)
    def _bn_fill():
        ones = jnp.ones((1, b.shape[1]), jnp.float32)
        bn_ref[:, pl.ds(j * N_TILE, N_TILE)] = lax.dot_general(
            ones, b * b, (((1,), (1,)), ((), ())),
            preferred_element_type=jnp.float32)

    bn = bn_ref[:, pl.ds(j * N_TILE, N_TILE)]
    qn = 0.25 * jnp.sum(s2 * s2, axis=1, keepdims=True)
    d2 = (bn + prod2) + qn
    r = jnp.sqrt(jnp.maximum(d2, 1e-12))

    last_j = N_PAD // N_TILE - 1

    @pl.when(j < last_j)
    def _store():
        r_ref[...] = r

    @pl.when(j == last_j)
    def _store_masked():
        col = lax.broadcasted_iota(jnp.int32, (Q_TILE, N_TILE), 1)
        r_ref[...] = jnp.where(col < N_REAL - last_j * N_TILE, r, T0)

    @pl.when(j == 0)
    def _mlp():
        s = -0.5 * s2
        h1 = jnp.maximum(
            lax.dot_general(s, w1_ref[...], (((1,), (0,)), ((), ())),
                            preferred_element_type=jnp.float32)
            + b1_ref[...], 0.0)
        h2 = jnp.maximum(
            lax.dot_general(h1, w2_ref[...], (((1,), (0,)), ((), ())),
                            preferred_element_type=jnp.float32)
            + b2_ref[...], 0.0)
        z = (lax.dot_general(h2, w3_ref[...], (((1,), (0,)), ((), ())),
                             preferred_element_type=jnp.float32)
             + b3_ref[...])
        neural_ref[...] = (1.0 - jax.nn.sigmoid(z[:, 0:1]))[:, 0]


def _distances_and_mlp(state, buf, w1, b1p, w2, b2p, w3, b3p):
    grid = (QC // Q_TILE, N_PAD // N_TILE)
    return pl.pallas_call(
        _dist_kernel,
        grid=grid,
        in_specs=[
            pl.BlockSpec((Q_TILE, 512), lambda i, j: (i, 0)),
            pl.BlockSpec((N_TILE, 512), lambda i, j: (j, 0)),
            pl.BlockSpec((512, 128), lambda i, j: (0, 0)),
            pl.BlockSpec((1, 128), lambda i, j: (0, 0)),
            pl.BlockSpec((128, 128), lambda i, j: (0, 0)),
            pl.BlockSpec((1, 128), lambda i, j: (0, 0)),
            pl.BlockSpec((128, 128), lambda i, j: (0, 0)),
            pl.BlockSpec((1, 128), lambda i, j: (0, 0)),
        ],
        out_specs=[
            pl.BlockSpec((Q_TILE, N_TILE), lambda i, j: (i, j)),
            pl.BlockSpec((Q_TILE,), lambda i, j: (i,)),
        ],
        out_shape=[
            jax.ShapeDtypeStruct((QC, N_PAD), jnp.float32),
            jax.ShapeDtypeStruct((QC,), jnp.float32),
        ],
        scratch_shapes=[pltpu.VMEM((1, N_PAD), jnp.float32)],
        compiler_params=pltpu.CompilerParams(
            dimension_semantics=("arbitrary", "arbitrary")),
    )(state, buf, w1, b1p, w2, b2p, w3, b3p)


def _extract_sums(cand_vregs):
    def body(_, carry):
        prev, cnt, s10, s15, s20, t, cb = carry
        w = jnp.full((L,), T0, jnp.float32)
        for v in cand_vregs:
            w = jnp.minimum(w, jnp.where(v > prev, v, T0))
        m = jnp.min(w)
        c = jnp.float32(0.0)
        for v in cand_vregs:
            c = c + jnp.sum(jnp.where(v == m, 1.0, 0.0))
        new_cnt = cnt + c
        crossing = jnp.logical_and(cnt < 20.0, new_cnt >= 20.0)
        t = jnp.where(crossing, m, t)
        cb = jnp.where(crossing, cnt, cb)
        s10 = s10 + m * jnp.clip(10.0 - cnt, 0.0, c)
        s15 = s15 + m * jnp.clip(15.0 - cnt, 0.0, c)
        s20 = s20 + m * jnp.clip(20.0 - cnt, 0.0, c)
        return m, new_cnt, s10, s15, s20, t, cb

    z = jnp.float32(0.0)
    init = (jnp.float32(-T0), z, z, z, z, jnp.float32(T0), z)
    return lax.fori_loop(0, 20, body, init)


def _sc_topk_kernel(r_hbm, neural_hbm, out_hbm,
                    buf0, buf1, cand, bmbuf, neu_v, out_v, sem0, sem1):
    wid = lax.axis_index("s") * NC + lax.axis_index("c")
    base = wid * ROWS_PER_W
    iota = lax.iota(jnp.int32, L)

    pltpu.sync_copy(neural_hbm.at[pl.ds(base, ROWS_PER_W)], neu_v)

    def reset_cand():
        for vi in range(CAND // L):
            cand[pl.ds(vi * L, L)] = jnp.full((L,), T0, jnp.float32)

    def compact(t_p):
        del t_p
        cvs = [cand[pl.ds(vi * L, L)] for vi in range(CAND // L)]
        _, _, _, _, _, t, _ = _extract_sums(cvs)
        q = jnp.int32(0)
        for v in cvs:
            mk = v < t
            plsc.store_compressed(cand.at[pl.ds(q, L)], v, mask=mk)
            q = q + jnp.sum(mk.astype(jnp.int32))
        n_t = jnp.int32(20) - q
        tv = jnp.full((L,), t, jnp.float32)
        for blk in range(2):
            mk = (iota + blk * L) < n_t
            plsc.store_compressed(cand.at[pl.ds(q + blk * L, L)], tv, mask=mk)
        q = q + n_t
        for vi in range(CAND // L):
            slot = iota + vi * L
            v = cand[pl.ds(vi * L, L)]
            cand[pl.ds(vi * L, L)] = jnp.where(slot < q, v, T0)
        return t, q

    def sub_insert(sv, t_p):
        t, p = t_p
        mks = [v < t for v in sv]
        cs = [plsc.all_reduce_population_count(mk)[0] for mk in mks]
        offs = []
        o = p
        for c in cs:
            offs.append(o)
            o = o + c
        for v, mk, off in zip(sv, mks, offs):
            plsc.store_compressed(cand.at[pl.ds(off, L)], v, mask=mk)
        return lax.cond(o >= COMPACT_AT, compact, lambda tp: tp, (t, o))

    def tree_min(vs):
        ms = list(vs)
        while len(ms) > 1:
            ms = [jnp.minimum(ms[i], ms[i + 1])
                  for i in range(0, len(ms), 2)]
        return ms[0]

    def scan_row(rowbuf):
        reset_cand()

        def pass_a(qi, _):
            for gq in range(4):
                gi = qi * 4 + gq
                b0 = gi * GROUP
                a0 = rowbuf[pl.ds(b0 * L, L)]
                a1 = rowbuf[pl.ds((b0 + 1) * L, L)]
                for u in range(2, GROUP, 2):
                    a0 = jnp.minimum(a0, rowbuf[pl.ds((b0 + u) * L, L)])
                    a1 = jnp.minimum(a1, rowbuf[pl.ds((b0 + u + 1) * L, L)])
                bmbuf[pl.ds(gi * L, L)] = jnp.minimum(a0, a1)
            return 0
        lax.fori_loop(0, NGROUPS // 4, pass_a, 0)

        tms = []
        for h in range(NFOLD):
            acc = bmbuf[pl.ds(h * 16 * L, L)]
            for u in range(1, 16):
                acc = jnp.minimum(acc, bmbuf[pl.ds((h * 16 + u) * L, L)])
            tms.append(acc)
        _, _, _, _, _, t_ub, _ = _extract_sums(tms)
        t_plus = lax.bitcast_convert_type(
            lax.bitcast_convert_type(t_ub, jnp.int32) + 1, jnp.float32)

        def pass_b(gi, t_p):
            bm = bmbuf[pl.ds(gi * L, L)]
            hit = plsc.all_reduce_population_count(bm < t_p[0])[0] > 0

            def rescan(t_p):
                for u in range(GROUP // BLK):
                    sv = [rowbuf[pl.ds((gi * GROUP + u * BLK + w) * L, L)]
                          for w in range(BLK)]
                    ms = tree_min(sv)
                    sub_hit = (plsc.all_reduce_population_count(
                        ms < t_p[0])[0] > 0)
                    t_p = lax.cond(sub_hit,
                                   functools.partial(sub_insert, sv),
                                   lambda tp: tp, t_p)
                return t_p

            return lax.cond(hit, rescan, lambda tp: tp, t_p)

        lax.fori_loop(0, NGROUPS, pass_b, (t_plus, jnp.int32(0)))
        cvs = [cand[pl.ds(vi * L, L)] for vi in range(CAND // L)]
        _, _, s10, s15, s20, _, _ = _extract_sums(cvs)
        return (s10 * 0.1 + s15 * (1.0 / 15.0) + s20 * 0.05) * (1.0 / 3.0)

    def wait(buf, sem):
        pltpu.make_async_copy(r_hbm.at[0], buf, sem).wait()

    for g in range(ROWS_PER_W // L):
        g0 = g * L
        pltpu.async_copy(r_hbm.at[base + g0], buf0, sem0)
        pltpu.async_copy(r_hbm.at[base + g0 + 1], buf1, sem1)

        def pair_body(pi, acc, g0=g0):
            r0 = g0 + 2 * pi

            wait(buf0, sem0)
            ens0 = scan_row(buf0)

            @pl.when(pi < L // 2 - 1)
            def _pf0():
                pltpu.async_copy(r_hbm.at[base + r0 + 2], buf0, sem0)

            wait(buf1, sem1)
            ens1 = scan_row(buf1)

            @pl.when(pi < L // 2 - 1)
            def _pf1():
                pltpu.async_copy(r_hbm.at[base + r0 + 3], buf1, sem1)

            acc = jnp.where(iota == 2 * pi, ens0, acc)
            acc = jnp.where(iota == 2 * pi + 1, ens1, acc)
            return acc

        acc = lax.fori_loop(0, L // 2, pair_body,
                            jnp.zeros((L,), jnp.float32))
        neu = neu_v[pl.ds(g * L, L)]
        out_v[pl.ds(g * L, L)] = 0.7 * acc + 0.3 * neu

    pltpu.sync_copy(out_v, out_hbm.at[pl.ds(base, ROWS_PER_W)])


def _sc_topk(r, neural):
    mesh = plsc.VectorSubcoreMesh(core_axis_name="c", subcore_axis_name="s")
    f = pl.kernel(
        _sc_topk_kernel,
        out_type=jax.ShapeDtypeStruct((QC,), jnp.float32),
        mesh=mesh,
        scratch_types=[
            pltpu.VMEM((N_PAD,), jnp.float32),
            pltpu.VMEM((N_PAD,), jnp.float32),
            pltpu.VMEM((CAND,), jnp.float32),
            pltpu.VMEM((NGROUPS * L,), jnp.float32),
            pltpu.VMEM((ROWS_PER_W,), jnp.float32),
            pltpu.VMEM((ROWS_PER_W,), jnp.float32),
            pltpu.SemaphoreType.DMA,
            pltpu.SemaphoreType.DMA,
        ],
        compiler_params=pltpu.CompilerParams(needs_layout_passes=False),
    )
    return f(r, neural)


@jax.jit
def kernel(state, state_buffer, W1, b1, W2, b2, W3, b3):
    N = state_buffer.shape[0]
    buf = jnp.pad(state_buffer, ((0, N_PAD - N), (0, 0)))
    b1p = b1[None, :]
    w2 = jnp.pad(W2, ((0, 0), (0, 128 - W2.shape[1])))
    b2p = jnp.pad(b2, (0, 128 - b2.shape[0]))[None, :]
    w3 = jnp.pad(W3, ((0, 128 - W3.shape[0]), (0, 128 - W3.shape[1])))
    b3p = jnp.pad(b3, (0, 128 - b3.shape[0]))[None, :]

    state2 = -2.0 * state
    outs = []
    for c in range(Q // QC):
        s2c = lax.slice_in_dim(state2, c * QC, (c + 1) * QC, axis=0)
        r, neural = _distances_and_mlp(s2c, buf, W1, b1p, w2, b2p, w3, b3p)
        outs.append(_sc_topk(r, neural))
    return jnp.concatenate(outs, axis=0)


# speedup vs baseline: 4.0236x; 1.0455x over previous
import functools

import jax
import jax.numpy as jnp
from jax import lax
from jax.experimental import pallas as pl
from jax.experimental.pallas import tpu as pltpu
from jax.experimental.pallas import tpu_sc as plsc

Q = 1024
QC = 256
Q_TILE = 128
N_TILE = 2048
N_PAD = 20480
N_REAL = 20000
BIG = 1.0e30
T0 = 3.0e38

NC, NS, L = 2, 16, 16
NW = NC * NS
ROWS_PER_W = QC // NW
VREGS_PER_ROW = N_PAD // L
CAND = 224
COMPACT_AT = 80
BLK = 8
GROUP = 16
NGROUPS = N_PAD // L // GROUP
NFOLD = NGROUPS // 16


def _dist_kernel(state2_ref, buf_ref, w1_ref, b1_ref, w2_ref, b2_ref,
                 w3_ref, b3_ref, r_ref, neural_ref, bn_ref):
    i = pl.program_id(0)
    j = pl.program_id(1)

    s2 = state2_ref[...]
    b = buf_ref[...]

    prod2 = lax.dot_general(s2, b, (((1,), (1,)), ((), ())),
                            preferred_element_type=jnp.float32)

    @pl.when(i == 0)
    def _bn_fill():
        ones = jnp.ones((1, b.shape[1]), jnp.float32)
        bn_ref[:, pl.ds(j * N_TILE, N_TILE)] = lax.dot_general(
            ones, b * b, (((1,), (1,)), ((), ())),
            preferred_element_type=jnp.float32)

    bn = bn_ref[:, pl.ds(j * N_TILE, N_TILE)]
    qn = 0.25 * jnp.sum(s2 * s2, axis=1, keepdims=True)
    d2 = (bn + prod2) + qn
    r = jnp.sqrt(jnp.maximum(d2, 1e-12))

    last_j = N_PAD // N_TILE - 1

    @pl.when(j < last_j)
    def _store():
        r_ref[...] = r

    @pl.when(j == last_j)
    def _store_masked():
        col = lax.broadcasted_iota(jnp.int32, (Q_TILE, N_TILE), 1)
        r_ref[...] = jnp.where(col < N_REAL - last_j * N_TILE, r, T0)

    @pl.when(j == 0)
    def _mlp():
        s = -0.5 * s2
        h1 = jnp.maximum(
            lax.dot_general(s, w1_ref[...], (((1,), (0,)), ((), ())),
                            preferred_element_type=jnp.float32)
            + b1_ref[...], 0.0)
        h2 = jnp.maximum(
            lax.dot_general(h1, w2_ref[...], (((1,), (0,)), ((), ())),
                            preferred_element_type=jnp.float32)
            + b2_ref[...], 0.0)
        z = (lax.dot_general(h2, w3_ref[...], (((1,), (0,)), ((), ())),
                             preferred_element_type=jnp.float32)
             + b3_ref[...])
        neural_ref[...] = (1.0 - jax.nn.sigmoid(z[:, 0:1]))[:, 0]


def _distances_and_mlp(state, buf, w1, b1p, w2, b2p, w3, b3p):
    grid = (QC // Q_TILE, N_PAD // N_TILE)
    return pl.pallas_call(
        _dist_kernel,
        grid=grid,
        in_specs=[
            pl.BlockSpec((Q_TILE, 512), lambda i, j: (i, 0)),
            pl.BlockSpec((N_TILE, 512), lambda i, j: (j, 0)),
            pl.BlockSpec((512, 128), lambda i, j: (0, 0)),
            pl.BlockSpec((1, 128), lambda i, j: (0, 0)),
            pl.BlockSpec((128, 128), lambda i, j: (0, 0)),
            pl.BlockSpec((1, 128), lambda i, j: (0, 0)),
            pl.BlockSpec((128, 128), lambda i, j: (0, 0)),
            pl.BlockSpec((1, 128), lambda i, j: (0, 0)),
        ],
        out_specs=[
            pl.BlockSpec((Q_TILE, N_TILE), lambda i, j: (i, j)),
            pl.BlockSpec((Q_TILE,), lambda i, j: (i,)),
        ],
        out_shape=[
            jax.ShapeDtypeStruct((QC, N_PAD), jnp.float32),
            jax.ShapeDtypeStruct((QC,), jnp.float32),
        ],
        scratch_shapes=[pltpu.VMEM((1, N_PAD), jnp.float32)],
        compiler_params=pltpu.CompilerParams(
            dimension_semantics=("arbitrary", "arbitrary")),
    )(state, buf, w1, b1p, w2, b2p, w3, b3p)


def _extract_sums(cand_vregs):
    def body(_, carry):
        prev, cnt, s10, s15, s20, t, cb = carry
        w = jnp.full((L,), T0, jnp.float32)
        for v in cand_vregs:
            w = jnp.minimum(w, jnp.where(v > prev, v, T0))
        m = jnp.min(w)
        c = jnp.float32(0.0)
        for v in cand_vregs:
            c = c + jnp.sum(jnp.where(v == m, 1.0, 0.0))
        new_cnt = cnt + c
        crossing = jnp.logical_and(cnt < 20.0, new_cnt >= 20.0)
        t = jnp.where(crossing, m, t)
        cb = jnp.where(crossing, cnt, cb)
        s10 = s10 + m * jnp.clip(10.0 - cnt, 0.0, c)
        s15 = s15 + m * jnp.clip(15.0 - cnt, 0.0, c)
        s20 = s20 + m * jnp.clip(20.0 - cnt, 0.0, c)
        return m, new_cnt, s10, s15, s20, t, cb

    z = jnp.float32(0.0)
    init = (jnp.float32(-T0), z, z, z, z, jnp.float32(T0), z)
    return lax.fori_loop(0, 20, body, init)


def _sc_topk_kernel(r_hbm, neural_hbm, out_hbm,
                    buf0, buf1, cand, bmbuf, neu_v, out_v, sem0, sem1):
    wid = lax.axis_index("s") * NC + lax.axis_index("c")
    base = wid * ROWS_PER_W
    iota = lax.iota(jnp.int32, L)

    pltpu.sync_copy(neural_hbm.at[pl.ds(base, ROWS_PER_W)],
                    neu_v.at[pl.ds(0, ROWS_PER_W)])

    def reset_cand():
        for vi in range(CAND // L):
            cand[pl.ds(vi * L, L)] = jnp.full((L,), T0, jnp.float32)

    def compact(t_p):
        del t_p
        cvs = [cand[pl.ds(vi * L, L)] for vi in range(CAND // L)]
        _, _, _, _, _, t, _ = _extract_sums(cvs)
        q = jnp.int32(0)
        for v in cvs:
            mk = v < t
            plsc.store_compressed(cand.at[pl.ds(q, L)], v, mask=mk)
            q = q + jnp.sum(mk.astype(jnp.int32))
        n_t = jnp.int32(20) - q
        tv = jnp.full((L,), t, jnp.float32)
        for blk in range(2):
            mk = (iota + blk * L) < n_t
            plsc.store_compressed(cand.at[pl.ds(q + blk * L, L)], tv, mask=mk)
        q = q + n_t
        for vi in range(CAND // L):
            slot = iota + vi * L
            v = cand[pl.ds(vi * L, L)]
            cand[pl.ds(vi * L, L)] = jnp.where(slot < q, v, T0)
        return t, q

    def sub_insert(sv, t_p):
        t, p = t_p
        mks = [v < t for v in sv]
        cs = [plsc.all_reduce_population_count(mk)[0] for mk in mks]
        offs = []
        o = p
        for c in cs:
            offs.append(o)
            o = o + c
        for v, mk, off in zip(sv, mks, offs):
            plsc.store_compressed(cand.at[pl.ds(off, L)], v, mask=mk)
        return lax.cond(o >= COMPACT_AT, compact, lambda tp: tp, (t, o))

    def tree_min(vs):
        ms = list(vs)
        while len(ms) > 1:
            ms = [jnp.minimum(ms[i], ms[i + 1])
                  for i in range(0, len(ms), 2)]
        return ms[0]

    def scan_row(rowbuf):
        reset_cand()

        def pass_a(qi, _):
            for gq in range(4):
                gi = qi * 4 + gq
                b0 = gi * GROUP
                a0 = rowbuf[pl.ds(b0 * L, L)]
                a1 = rowbuf[pl.ds((b0 + 1) * L, L)]
                for u in range(2, GROUP, 2):
                    a0 = jnp.minimum(a0, rowbuf[pl.ds((b0 + u) * L, L)])
                    a1 = jnp.minimum(a1, rowbuf[pl.ds((b0 + u + 1) * L, L)])
                bmbuf[pl.ds(gi * L, L)] = jnp.minimum(a0, a1)
            return 0
        lax.fori_loop(0, NGROUPS // 4, pass_a, 0)

        tms = []
        for h in range(NFOLD):
            acc = bmbuf[pl.ds(h * 16 * L, L)]
            for u in range(1, 16):
                acc = jnp.minimum(acc, bmbuf[pl.ds((h * 16 + u) * L, L)])
            tms.append(acc)
        _, _, _, _, _, t_ub, _ = _extract_sums(tms)
        t_plus = lax.bitcast_convert_type(
            lax.bitcast_convert_type(t_ub, jnp.int32) + 1, jnp.float32)

        def pass_b(gi, t_p):
            bm = bmbuf[pl.ds(gi * L, L)]
            hit = plsc.all_reduce_population_count(bm < t_p[0])[0] > 0

            def rescan(t_p):
                for u in range(GROUP // BLK):
                    sv = [rowbuf[pl.ds((gi * GROUP + u * BLK + w) * L, L)]
                          for w in range(BLK)]
                    ms = tree_min(sv)
                    sub_hit = (plsc.all_reduce_population_count(
                        ms < t_p[0])[0] > 0)
                    t_p = lax.cond(sub_hit,
                                   functools.partial(sub_insert, sv),
                                   lambda tp: tp, t_p)
                return t_p

            return lax.cond(hit, rescan, lambda tp: tp, t_p)

        lax.fori_loop(0, NGROUPS, pass_b, (t_plus, jnp.int32(0)))
        cvs = [cand[pl.ds(vi * L, L)] for vi in range(CAND // L)]
        _, _, s10, s15, s20, _, _ = _extract_sums(cvs)
        return (s10 * 0.1 + s15 * (1.0 / 15.0) + s20 * 0.05) * (1.0 / 3.0)

    def wait(buf, sem):
        pltpu.make_async_copy(r_hbm.at[0], buf, sem).wait()

    npairs = ROWS_PER_W // 2
    pltpu.async_copy(r_hbm.at[base], buf0, sem0)
    pltpu.async_copy(r_hbm.at[base + 1], buf1, sem1)

    def pair_body(pi, acc):
        r0 = 2 * pi

        wait(buf0, sem0)
        ens0 = scan_row(buf0)

        @pl.when(pi < npairs - 1)
        def _pf0():
            pltpu.async_copy(r_hbm.at[base + r0 + 2], buf0, sem0)

        wait(buf1, sem1)
        ens1 = scan_row(buf1)

        @pl.when(pi < npairs - 1)
        def _pf1():
            pltpu.async_copy(r_hbm.at[base + r0 + 3], buf1, sem1)

        acc = jnp.where(iota == r0, ens0, acc)
        acc = jnp.where(iota == r0 + 1, ens1, acc)
        return acc

    acc = lax.fori_loop(0, npairs, pair_body,
                        jnp.zeros((L,), jnp.float32))
    neu = neu_v[pl.ds(0, L)]
    out_v[pl.ds(0, L)] = 0.7 * acc + 0.3 * neu

    pltpu.sync_copy(out_v.at[pl.ds(0, ROWS_PER_W)],
                    out_hbm.at[pl.ds(base, ROWS_PER_W)])


def _sc_topk(r, neural):
    mesh = plsc.VectorSubcoreMesh(core_axis_name="c", subcore_axis_name="s")
    f = pl.kernel(
        _sc_topk_kernel,
        out_type=jax.ShapeDtypeStruct((QC,), jnp.float32),
        mesh=mesh,
        scratch_types=[
            pltpu.VMEM((N_PAD,), jnp.float32),
            pltpu.VMEM((N_PAD,), jnp.float32),
            pltpu.VMEM((CAND,), jnp.float32),
            pltpu.VMEM((NGROUPS * L,), jnp.float32),
            pltpu.VMEM((L,), jnp.float32),
            pltpu.VMEM((L,), jnp.float32),
            pltpu.SemaphoreType.DMA,
            pltpu.SemaphoreType.DMA,
        ],
        compiler_params=pltpu.CompilerParams(needs_layout_passes=False),
    )
    return f(r, neural)


@jax.jit
def kernel(state, state_buffer, W1, b1, W2, b2, W3, b3):
    N = state_buffer.shape[0]
    buf = jnp.pad(state_buffer, ((0, N_PAD - N), (0, 0)))
    b1p = b1[None, :]
    w2 = jnp.pad(W2, ((0, 0), (0, 128 - W2.shape[1])))
    b2p = jnp.pad(b2, (0, 128 - b2.shape[0]))[None, :]
    w3 = jnp.pad(W3, ((0, 128 - W3.shape[0]), (0, 128 - W3.shape[1])))
    b3p = jnp.pad(b3, (0, 128 - b3.shape[0]))[None, :]

    state2 = -2.0 * state
    outs = []
    for c in range(Q // QC):
        s2c = lax.slice_in_dim(state2, c * QC, (c + 1) * QC, axis=0)
        r, neural = _distances_and_mlp(s2c, buf, W1, b1p, w2, b2p, w3, b3p)
        outs.append(_sc_topk(r, neural))
    return jnp.concatenate(outs, axis=0)
